# baseline (speedup 1.0000x reference)
import functools

import jax
import jax.numpy as jnp
from jax import lax
from jax.experimental import pallas as pl
from jax.experimental.pallas import tpu as pltpu
from jax.experimental.pallas import tpu_sc as plsc

EMB = 64
MAX_LEN = 200
BATCH = 4096
NUM_WORKERS = 32
BB = BATCH // NUM_WORKERS


def _body(idxT_hbm, table_hbm, pos_hbm, out_hbm,
          idx_v, pos_v, rows_a, rows_b, stage_v, sem_a, sem_b):
    wid = lax.axis_index("s") * 2 + lax.axis_index("c")
    pltpu.sync_copy(pos_hbm, pos_v)
    pltpu.sync_copy(idxT_hbm.at[:, pl.ds(wid * BB, BB)], idx_v)
    iota = lax.iota(jnp.int32, 16)

    def process(l, rows_v):
        for half in range(EMB // 16):
            pv16 = pos_v[l, pl.ds(half * 16, 16)]
            for ei in range(16):
                e = half * 16 + ei
                tr, er = e // 8, e % 8
                pv = jnp.full((16,), pv16[ei], jnp.float32)
                col = jnp.full((16,), e, jnp.int32)
                for bg in range(8):
                    v = plsc.load_gather(rows_v, [bg * 16 + iota, col])
                    stage_v[tr, er, pl.ds(bg * 16, 16)] = v + pv

        pltpu.sync_copy(stage_v, out_hbm.at[l, :, wid])

    pltpu.async_copy(table_hbm.at[idx_v.at[0]], rows_a, sem_a)
    pltpu.async_copy(table_hbm.at[idx_v.at[1]], rows_b, sem_b)

    def loop_body(i, _):
        for rows_v, sem, off in ((rows_a, sem_a, 0), (rows_b, sem_b, 1)):
            l = 2 * i + off
            pltpu.make_async_copy(table_hbm.at[idx_v.at[l]], rows_v, sem).wait()
            process(l, rows_v)

            @pl.when(l + 2 < MAX_LEN)
            def _():
                pltpu.async_copy(table_hbm.at[idx_v.at[l + 2]], rows_v, sem)

        return 0

    lax.fori_loop(0, MAX_LEN // 2, loop_body, 0)


def kernel(to_emb, token_table, pos_table):
    idxT = to_emb.T.astype(jnp.int32)

    mesh = plsc.VectorSubcoreMesh(core_axis_name="c", subcore_axis_name="s")
    k = functools.partial(
        pl.kernel,
        out_type=jax.ShapeDtypeStruct((MAX_LEN, 8, NUM_WORKERS, 8, BB), jnp.float32),
        mesh=mesh,
        scratch_types=[
            pltpu.VMEM((MAX_LEN, BB), jnp.int32),
            pltpu.VMEM((MAX_LEN, EMB), jnp.float32),
            pltpu.VMEM((BB, EMB), jnp.float32),
            pltpu.VMEM((BB, EMB), jnp.float32),
            pltpu.VMEM((8, 8, BB), jnp.float32),
            pltpu.SemaphoreType.DMA,
            pltpu.SemaphoreType.DMA,
        ],
        compiler_params=pltpu.CompilerParams(
            use_tc_tiling_on_sc=False, needs_layout_passes=False
        ),
    )(_body)
    out5 = k(idxT, token_table, pos_table)
    return out5.transpose((2, 4, 0, 1, 3)).reshape(BATCH, MAX_LEN, EMB)


# speedup vs baseline: 1.8299x; 1.8299x over previous
import functools

import jax
import jax.numpy as jnp
from jax import lax
from jax.experimental import pallas as pl
from jax.experimental.pallas import tpu as pltpu
from jax.experimental.pallas import tpu_sc as plsc

EMB = 64
MAX_LEN = 200
BATCH = 4096
NUM_WORKERS = 32
BB = BATCH // NUM_WORKERS


def _body(idxT_hbm, table_hbm, pos_hbm, out_hbm,
          idx_v, pos_v, rows_a, rows_b, stage_v, sem_a, sem_b):
    wid = lax.axis_index("s") * 2 + lax.axis_index("c")
    pltpu.sync_copy(pos_hbm, pos_v)
    pltpu.sync_copy(idxT_hbm.at[:, pl.ds(wid * BB, BB)], idx_v)
    iota = lax.iota(jnp.int32, 16)

    def process(l, rows_v):
        lvec = jnp.full((16,), 0, jnp.int32) + l

        def d_body(d, _):
            for h in range(EMB // 16):
                diag = ((iota + d) & 15) + 16 * h
                tr_i = diag >> 3
                er_i = diag & 7
                pv = plsc.load_gather(pos_v, [lvec, diag])
                for bg in range(8):
                    b_i = bg * 16 + iota
                    v = plsc.load_gather(rows_v, [b_i, diag])
                    plsc.store_scatter(stage_v, [tr_i, er_i, b_i], v + pv)
            return 0

        lax.fori_loop(0, 16, d_body, 0)
        pltpu.sync_copy(stage_v, out_hbm.at[l, :, wid])

    pltpu.async_copy(table_hbm.at[idx_v.at[0]], rows_a, sem_a)
    pltpu.async_copy(table_hbm.at[idx_v.at[1]], rows_b, sem_b)

    def loop_body(i, _):
        for rows_v, sem, off in ((rows_a, sem_a, 0), (rows_b, sem_b, 1)):
            l = 2 * i + off
            pltpu.make_async_copy(table_hbm.at[idx_v.at[l]], rows_v, sem).wait()
            process(l, rows_v)

            @pl.when(l + 2 < MAX_LEN)
            def _():
                pltpu.async_copy(table_hbm.at[idx_v.at[l + 2]], rows_v, sem)

        return 0

    lax.fori_loop(0, MAX_LEN // 2, loop_body, 0)


def kernel(to_emb, token_table, pos_table):
    idxT = to_emb.T.astype(jnp.int32)

    mesh = plsc.VectorSubcoreMesh(core_axis_name="c", subcore_axis_name="s")
    k = functools.partial(
        pl.kernel,
        out_type=jax.ShapeDtypeStruct((MAX_LEN, 8, NUM_WORKERS, 8, BB), jnp.float32),
        mesh=mesh,
        scratch_types=[
            pltpu.VMEM((MAX_LEN, BB), jnp.int32),
            pltpu.VMEM((MAX_LEN, EMB), jnp.float32),
            pltpu.VMEM((BB, EMB), jnp.float32),
            pltpu.VMEM((BB, EMB), jnp.float32),
            pltpu.VMEM((8, 8, BB), jnp.float32),
            pltpu.SemaphoreType.DMA,
            pltpu.SemaphoreType.DMA,
        ],
        compiler_params=pltpu.CompilerParams(
            use_tc_tiling_on_sc=False, needs_layout_passes=False
        ),
    )(_body)
    out5 = k(idxT, token_table, pos_table)
    return out5.transpose((2, 4, 0, 1, 3)).reshape(BATCH, MAX_LEN, EMB)


# speedup vs baseline: 1.9398x; 1.0601x over previous
import functools

import jax
import jax.numpy as jnp
from jax import lax
from jax.experimental import pallas as pl
from jax.experimental.pallas import tpu as pltpu
from jax.experimental.pallas import tpu_sc as plsc

EMB = 64
MAX_LEN = 200
BATCH = 4096
NUM_WORKERS = 32
BB = BATCH // NUM_WORKERS


def _body(idxT_hbm, table_hbm, pos_hbm, out_hbm,
          idx_v, pos_v, rows_a, rows_b, stage_a, stage_b,
          sem_a, sem_b, sem_oa, sem_ob):
    wid = lax.axis_index("s") * 2 + lax.axis_index("c")
    pltpu.sync_copy(pos_hbm, pos_v)
    pltpu.sync_copy(idxT_hbm.at[:, pl.ds(wid * BB, BB)], idx_v)
    iota = lax.iota(jnp.int32, 16)

    def process(i, l, rows_v, stage_v, osem):
        @pl.when(i > 0)
        def _():
            pltpu.make_async_copy(stage_v, out_hbm.at[l, :, wid], osem).wait()

        lvec = jnp.full((16,), 0, jnp.int32) + l

        def d_body(d, _):
            for h in range(EMB // 16):
                diag = ((iota + d) & 15) + 16 * h
                tr_i = diag >> 3
                er_i = diag & 7
                pv = plsc.load_gather(pos_v, [lvec, diag])
                for bg in range(8):
                    b_i = bg * 16 + iota
                    v = plsc.load_gather(rows_v, [b_i, diag])
                    plsc.store_scatter(stage_v, [tr_i, er_i, b_i], v + pv)
            return 0

        lax.fori_loop(0, 16, d_body, 0)
        pltpu.async_copy(stage_v, out_hbm.at[l, :, wid], osem)

    pltpu.async_copy(table_hbm.at[idx_v.at[0]], rows_a, sem_a)
    pltpu.async_copy(table_hbm.at[idx_v.at[1]], rows_b, sem_b)

    def loop_body(i, _):
        for rows_v, sem, stage_v, osem, off in (
            (rows_a, sem_a, stage_a, sem_oa, 0),
            (rows_b, sem_b, stage_b, sem_ob, 1),
        ):
            l = 2 * i + off
            pltpu.make_async_copy(table_hbm.at[idx_v.at[l]], rows_v, sem).wait()
            process(i, l, rows_v, stage_v, osem)

            @pl.when(l + 2 < MAX_LEN)
            def _():
                pltpu.async_copy(table_hbm.at[idx_v.at[l + 2]], rows_v, sem)

        return 0

    lax.fori_loop(0, MAX_LEN // 2, loop_body, 0)
    pltpu.make_async_copy(stage_a, out_hbm.at[MAX_LEN - 2, :, wid], sem_oa).wait()
    pltpu.make_async_copy(stage_b, out_hbm.at[MAX_LEN - 1, :, wid], sem_ob).wait()


def kernel(to_emb, token_table, pos_table):
    idxT = to_emb.T.astype(jnp.int32)

    mesh = plsc.VectorSubcoreMesh(core_axis_name="c", subcore_axis_name="s")
    k = functools.partial(
        pl.kernel,
        out_type=jax.ShapeDtypeStruct((MAX_LEN, 8, NUM_WORKERS, 8, BB), jnp.float32),
        mesh=mesh,
        scratch_types=[
            pltpu.VMEM((MAX_LEN, BB), jnp.int32),
            pltpu.VMEM((MAX_LEN, EMB), jnp.float32),
            pltpu.VMEM((BB, EMB), jnp.float32),
            pltpu.VMEM((BB, EMB), jnp.float32),
            pltpu.VMEM((8, 8, BB), jnp.float32),
            pltpu.VMEM((8, 8, BB), jnp.float32),
            pltpu.SemaphoreType.DMA,
            pltpu.SemaphoreType.DMA,
            pltpu.SemaphoreType.DMA,
            pltpu.SemaphoreType.DMA,
        ],
        compiler_params=pltpu.CompilerParams(
            use_tc_tiling_on_sc=False, needs_layout_passes=False
        ),
    )(_body)
    out5 = k(idxT, token_table, pos_table)
    return out5.transpose((2, 4, 0, 1, 3)).reshape(BATCH, MAX_LEN, EMB)


# speedup vs baseline: 2.5441x; 1.3115x over previous
import functools

import jax
import jax.numpy as jnp
from jax import lax
from jax.experimental import pallas as pl
from jax.experimental.pallas import tpu as pltpu
from jax.experimental.pallas import tpu_sc as plsc

EMB = 64
MAX_LEN = 200
BATCH = 4096
NUM_WORKERS = 32
BB = BATCH // NUM_WORKERS


def _body(idxT_hbm, table_hbm, pos_hbm, out_hbm,
          idx_v, pos_v, rows_a, rows_b, stage_a, stage_b,
          sem_a, sem_b, sem_oa, sem_ob):
    wid = lax.axis_index("s") * 2 + lax.axis_index("c")
    pltpu.sync_copy(pos_hbm, pos_v)
    pltpu.sync_copy(idxT_hbm.at[:, pl.ds(wid * BB, BB)], idx_v)
    iota = lax.iota(jnp.int32, 16)

    def process(i, l, rows_v, stage_v, osem):
        @pl.when(i > 0)
        def _():
            pltpu.make_async_copy(stage_v, out_hbm.at[l, :, wid], osem).wait()

        lvec = jnp.full((16,), 0, jnp.int32) + l

        @plsc.parallel_loop(0, 16, 1, unroll=4)
        def d_body(d):
            for h in range(EMB // 16):
                diag = ((iota + d) & 15) + 16 * h
                tr_i = diag >> 3
                er_i = diag & 7
                pv = plsc.load_gather(pos_v, [lvec, diag])
                for bg in range(8):
                    b_i = bg * 16 + iota
                    v = plsc.load_gather(rows_v, [b_i, diag])
                    plsc.store_scatter(stage_v, [tr_i, er_i, b_i], v + pv)
        pltpu.async_copy(stage_v, out_hbm.at[l, :, wid], osem)

    pltpu.async_copy(table_hbm.at[idx_v.at[0]], rows_a, sem_a)
    pltpu.async_copy(table_hbm.at[idx_v.at[1]], rows_b, sem_b)

    def loop_body(i, _):
        for rows_v, sem, stage_v, osem, off in (
            (rows_a, sem_a, stage_a, sem_oa, 0),
            (rows_b, sem_b, stage_b, sem_ob, 1),
        ):
            l = 2 * i + off
            pltpu.make_async_copy(table_hbm.at[idx_v.at[l]], rows_v, sem).wait()
            process(i, l, rows_v, stage_v, osem)

            @pl.when(l + 2 < MAX_LEN)
            def _():
                pltpu.async_copy(table_hbm.at[idx_v.at[l + 2]], rows_v, sem)

        return 0

    lax.fori_loop(0, MAX_LEN // 2, loop_body, 0)
    pltpu.make_async_copy(stage_a, out_hbm.at[MAX_LEN - 2, :, wid], sem_oa).wait()
    pltpu.make_async_copy(stage_b, out_hbm.at[MAX_LEN - 1, :, wid], sem_ob).wait()


def kernel(to_emb, token_table, pos_table):
    idxT = to_emb.T.astype(jnp.int32)

    mesh = plsc.VectorSubcoreMesh(core_axis_name="c", subcore_axis_name="s")
    k = functools.partial(
        pl.kernel,
        out_type=jax.ShapeDtypeStruct((MAX_LEN, 8, NUM_WORKERS, 8, BB), jnp.float32),
        mesh=mesh,
        scratch_types=[
            pltpu.VMEM((MAX_LEN, BB), jnp.int32),
            pltpu.VMEM((MAX_LEN, EMB), jnp.float32),
            pltpu.VMEM((BB, EMB), jnp.float32),
            pltpu.VMEM((BB, EMB), jnp.float32),
            pltpu.VMEM((8, 8, BB), jnp.float32),
            pltpu.VMEM((8, 8, BB), jnp.float32),
            pltpu.SemaphoreType.DMA,
            pltpu.SemaphoreType.DMA,
            pltpu.SemaphoreType.DMA,
            pltpu.SemaphoreType.DMA,
        ],
        compiler_params=pltpu.CompilerParams(
            use_tc_tiling_on_sc=False, needs_layout_passes=False
        ),
    )(_body)
    out5 = k(idxT, token_table, pos_table)
    return out5.transpose((2, 4, 0, 1, 3)).reshape(BATCH, MAX_LEN, EMB)


# speedup vs baseline: 3.0624x; 1.2037x over previous
import functools

import jax
import jax.numpy as jnp
from jax import lax
from jax.experimental import pallas as pl
from jax.experimental.pallas import tpu as pltpu
from jax.experimental.pallas import tpu_sc as plsc

EMB = 64
MAX_LEN = 200
BATCH = 4096
NUM_WORKERS = 32
BB = BATCH // NUM_WORKERS


def _body(idxT_hbm, table_hbm, pos_hbm, out_hbm,
          idx_v, pos_v, rows_a, rows_b, stage_a, stage_b,
          sem_a, sem_b, sem_oa, sem_ob):
    wid = lax.axis_index("s") * 2 + lax.axis_index("c")
    pltpu.sync_copy(pos_hbm, pos_v)
    pltpu.sync_copy(idxT_hbm.at[:, pl.ds(wid * BB, BB)], idx_v)
    iota = lax.iota(jnp.int32, 16)

    def fetch(l, rows_v, sem):
        def enq(bg, _):
            v16 = idx_v[l, pl.ds(bg * 16, 16)]
            for j in range(16):
                pltpu.async_copy(
                    table_hbm.at[v16[j]], rows_v.at[bg * 16 + j], sem
                )
            return 0

        lax.fori_loop(0, BB // 16, enq, 0)

    def process(i, l, rows_v, stage_v, osem):
        @pl.when(i > 0)
        def _():
            pltpu.make_async_copy(stage_v, out_hbm.at[l, :, wid], osem).wait()

        lvec = jnp.full((16,), 0, jnp.int32) + l

        @plsc.parallel_loop(0, 16, 1, unroll=4)
        def d_body(d):
            for h in range(EMB // 16):
                diag = ((iota + d) & 15) + 16 * h
                tr_i = diag >> 3
                er_i = diag & 7
                pv = plsc.load_gather(pos_v, [lvec, diag])
                for bg in range(8):
                    b_i = bg * 16 + iota
                    v = plsc.load_gather(rows_v, [b_i, diag])
                    plsc.store_scatter(stage_v, [tr_i, er_i, b_i], v + pv)

        pltpu.async_copy(stage_v, out_hbm.at[l, :, wid], osem)

    fetch(0, rows_a, sem_a)
    fetch(1, rows_b, sem_b)

    def loop_body(i, _):
        for rows_v, sem, stage_v, osem, off in (
            (rows_a, sem_a, stage_a, sem_oa, 0),
            (rows_b, sem_b, stage_b, sem_ob, 1),
        ):
            l = 2 * i + off
            pltpu.make_async_copy(table_hbm.at[pl.ds(0, BB)], rows_v, sem).wait()
            process(i, l, rows_v, stage_v, osem)

            @pl.when(l + 2 < MAX_LEN)
            def _():
                fetch(l + 2, rows_v, sem)

        return 0

    lax.fori_loop(0, MAX_LEN // 2, loop_body, 0)
    pltpu.make_async_copy(stage_a, out_hbm.at[MAX_LEN - 2, :, wid], sem_oa).wait()
    pltpu.make_async_copy(stage_b, out_hbm.at[MAX_LEN - 1, :, wid], sem_ob).wait()


def kernel(to_emb, token_table, pos_table):
    idxT = to_emb.T.astype(jnp.int32)

    mesh = plsc.VectorSubcoreMesh(core_axis_name="c", subcore_axis_name="s")
    k = functools.partial(
        pl.kernel,
        out_type=jax.ShapeDtypeStruct((MAX_LEN, 8, NUM_WORKERS, 8, BB), jnp.float32),
        mesh=mesh,
        scratch_types=[
            pltpu.VMEM((MAX_LEN, BB), jnp.int32),
            pltpu.VMEM((MAX_LEN, EMB), jnp.float32),
            pltpu.VMEM((BB, EMB), jnp.float32),
            pltpu.VMEM((BB, EMB), jnp.float32),
            pltpu.VMEM((8, 8, BB), jnp.float32),
            pltpu.VMEM((8, 8, BB), jnp.float32),
            pltpu.SemaphoreType.DMA,
            pltpu.SemaphoreType.DMA,
            pltpu.SemaphoreType.DMA,
            pltpu.SemaphoreType.DMA,
        ],
        compiler_params=pltpu.CompilerParams(
            use_tc_tiling_on_sc=True, needs_layout_passes=False
        ),
    )(_body)
    out5 = k(idxT, token_table, pos_table)
    return out5.transpose((2, 4, 0, 1, 3)).reshape(BATCH, MAX_LEN, EMB)
